# baseline (speedup 1.0000x reference)
import functools

import jax
import jax.numpy as jnp
from jax import lax
from jax.experimental import pallas as pl
from jax.experimental.pallas import tpu as pltpu
from jax.experimental.pallas import tpu_sc as plsc

_N_ROWS = 128
_N_COLS = 128
_N_ACT = 1024
_BATCH = 16384

_NC = 2
_NS = 16
_L = 16
_NW = _NC * _NS
_BPW = _BATCH // _NW
_K = 16
_NG = _BPW // _K
_NCHAIN = 4
_SEG = _N_ACT // _NCHAIN


def _sc_body(obs_hbm, tab_hbm, out_hbm, obs_v, idx_v, buf0, buf1, res_v,
             sem0, sem1):
    wid = lax.axis_index("s") * _NC + lax.axis_index("c")
    base = wid * _BPW

    pltpu.sync_copy(obs_hbm.at[pl.ds(base * 2, _BPW * 2)], obs_v)

    iota = lax.iota(jnp.int32, _L)
    zeros = jnp.zeros((_L,), jnp.int32)

    for g in range(_NG):
        rsel = (g * _L + iota) * 2
        r = plsc.load_gather(obs_v, [rsel])
        c = plsc.load_gather(obs_v, [rsel + 1])
        idx_v[g, :] = r * _N_COLS + c

    bufs = (buf0, buf1)
    sems = (sem0, sem1)

    handles = [None] * _NG
    handles[0] = pltpu.make_async_copy(tab_hbm.at[idx_v.at[0]], buf0, sem0)
    handles[0].start()

    neg_inf = jnp.full((_L,), -jnp.inf, jnp.float32)

    for g in range(_NG):
        if g + 1 < _NG:
            nb = (g + 1) % 2
            handles[g + 1] = pltpu.make_async_copy(
                tab_hbm.at[idx_v.at[g + 1]], bufs[nb], sems[nb])
            handles[g + 1].start()
        buf = bufs[g % 2]
        handles[g].wait()

        def step(_, carry, buf=buf):
            out = []
            for k in range(_NCHAIN):
                bv, bi, col = carry[k]
                v = plsc.load_gather(buf, [iota, col])
                m = v > bv
                bv = jnp.where(m, v, bv)
                bi = jnp.where(m, col, bi)
                out.append((bv, bi, col + 1))
            return tuple(out)

        init = tuple(
            (neg_inf, zeros, jnp.full((_L,), k * _SEG, jnp.int32))
            for k in range(_NCHAIN))
        fin = lax.fori_loop(0, _SEG, step, init)

        bv, bi = fin[0][0], fin[0][1]
        for k in range(1, _NCHAIN):
            v2, i2 = fin[k][0], fin[k][1]
            m = v2 > bv
            bv = jnp.where(m, v2, bv)
            bi = jnp.where(m, i2, bi)
        res_v[g, :] = bi

    pltpu.sync_copy(res_v, out_hbm.at[pl.ds(wid * _NG, _NG)])


@functools.partial(jax.jit, static_argnums=())
def _run(obs, tab):
    fn = pl.kernel(
        _sc_body,
        out_type=jax.ShapeDtypeStruct((_NW * _NG, _L), jnp.int32),
        mesh=plsc.VectorSubcoreMesh(core_axis_name="c", subcore_axis_name="s"),
        compiler_params=pltpu.CompilerParams(needs_layout_passes=False),
        scratch_types=[
            pltpu.VMEM((_BPW * 2,), jnp.int32),
            pltpu.VMEM((_NG, _L), jnp.int32),
            pltpu.VMEM((_K, _N_ACT), jnp.float32),
            pltpu.VMEM((_K, _N_ACT), jnp.float32),
            pltpu.VMEM((_NG, _L), jnp.int32),
            pltpu.SemaphoreType.DMA,
            pltpu.SemaphoreType.DMA,
        ],
    )
    return fn(obs, tab)


def kernel(observation, q_table):
    obs = observation.astype(jnp.int32).reshape(_BATCH * 2)
    tab = q_table.reshape(_N_ROWS * _N_COLS, _N_ACT)
    out = _run(obs, tab)
    return out.reshape(_BATCH)


# speedup vs baseline: 1.1745x; 1.1745x over previous
import functools

import jax
import jax.numpy as jnp
from jax import lax
from jax.experimental import pallas as pl
from jax.experimental.pallas import tpu as pltpu
from jax.experimental.pallas import tpu_sc as plsc

_N_ROWS = 128
_N_COLS = 128
_N_ACT = 1024
_BATCH = 16384

_NC = 2
_NS = 16
_L = 16
_NW = _NC * _NS
_BPW = _BATCH // _NW
_K = 16
_NG = _BPW // _K
_NCHAIN = 4
_SEG = _N_ACT // _NCHAIN
_UNROLL = 8


def _sc_body(obs_hbm, tab_hbm, out_hbm, obs_v, idx_v, buf0, buf1, res_v,
             sem0, sem1):
    wid = lax.axis_index("s") * _NC + lax.axis_index("c")
    base = wid * _BPW

    pltpu.sync_copy(obs_hbm.at[pl.ds(base * 2, _BPW * 2)], obs_v)

    iota = lax.iota(jnp.int32, _L)
    zeros = jnp.zeros((_L,), jnp.int32)

    for g in range(_NG):
        rsel = (g * _L + iota) * 2
        r = plsc.load_gather(obs_v, [rsel])
        c = plsc.load_gather(obs_v, [rsel + 1])
        idx_v[g, :] = r * _N_COLS + c

    bufs = (buf0, buf1)
    sems = (sem0, sem1)

    handles = [None] * _NG
    handles[0] = pltpu.make_async_copy(tab_hbm.at[idx_v.at[0]], buf0, sem0)
    handles[0].start()

    neg_inf = jnp.full((_L,), -jnp.inf, jnp.float32)

    for g in range(_NG):
        if g + 1 < _NG:
            nb = (g + 1) % 2
            handles[g + 1] = pltpu.make_async_copy(
                tab_hbm.at[idx_v.at[g + 1]], bufs[nb], sems[nb])
            handles[g + 1].start()
        buf = bufs[g % 2]
        handles[g].wait()

        def step(_, carry, buf=buf):
            out = []
            for k in range(_NCHAIN):
                bv, bi, col = carry[k]
                v = plsc.load_gather(buf, [iota, col])
                m = v > bv
                bv = jnp.where(m, v, bv)
                bi = jnp.where(m, col, bi)
                out.append((bv, bi, col + 1))
            return tuple(out)

        init = tuple(
            (neg_inf, zeros, jnp.full((_L,), k * _SEG, jnp.int32))
            for k in range(_NCHAIN))
        fin = lax.fori_loop(0, _SEG // _UNROLL,
                            lambda i, c: functools.reduce(
                                lambda cc, _: step(i, cc), range(_UNROLL), c),
                            init)

        bv, bi = fin[0][0], fin[0][1]
        for k in range(1, _NCHAIN):
            v2, i2 = fin[k][0], fin[k][1]
            m = v2 > bv
            bv = jnp.where(m, v2, bv)
            bi = jnp.where(m, i2, bi)
        res_v[g, :] = bi

    pltpu.sync_copy(res_v, out_hbm.at[pl.ds(wid * _NG, _NG)])


@functools.partial(jax.jit, static_argnums=())
def _run(obs, tab):
    fn = pl.kernel(
        _sc_body,
        out_type=jax.ShapeDtypeStruct((_NW * _NG, _L), jnp.int32),
        mesh=plsc.VectorSubcoreMesh(core_axis_name="c", subcore_axis_name="s"),
        compiler_params=pltpu.CompilerParams(needs_layout_passes=False),
        scratch_types=[
            pltpu.VMEM((_BPW * 2,), jnp.int32),
            pltpu.VMEM((_NG, _L), jnp.int32),
            pltpu.VMEM((_K, _N_ACT), jnp.float32),
            pltpu.VMEM((_K, _N_ACT), jnp.float32),
            pltpu.VMEM((_NG, _L), jnp.int32),
            pltpu.SemaphoreType.DMA,
            pltpu.SemaphoreType.DMA,
        ],
    )
    return fn(obs, tab)


def kernel(observation, q_table):
    obs = observation.astype(jnp.int32).reshape(_BATCH * 2)
    tab = q_table.reshape(_N_ROWS * _N_COLS, _N_ACT)
    out = _run(obs, tab)
    return out.reshape(_BATCH)


# speedup vs baseline: 5.1860x; 4.4154x over previous
import jax
import jax.numpy as jnp
from jax import lax
from jax.experimental import pallas as pl
from jax.experimental.pallas import tpu as pltpu
from jax.experimental.pallas import tpu_sc as plsc

_N_ROWS = 128
_N_COLS = 128
_N_ACT = 1024
_BATCH = 16384

_NC = 2
_NS = 16
_L = 16
_NW = _NC * _NS
_BPW = _BATCH // _NW
_K = 16
_NG = _BPW // _K
_NROWCHAIN = 4
_STEPS = _N_ACT // _L
_UNROLL = 4


def _sc_body(obs_hbm, tab_hbm, out_hbm, obs_v, idx_v, buf0, buf1,
             cval_v, cidx_v, res_v, sem0, sem1):
    wid = lax.axis_index("s") * _NC + lax.axis_index("c")
    base = wid * _BPW

    pltpu.sync_copy(obs_hbm.at[pl.ds(base * 2, _BPW * 2)], obs_v)

    iota = lax.iota(jnp.int32, _L)

    for g in range(_NG):
        rsel = (g * _L + iota) * 2
        r = plsc.load_gather(obs_v, [rsel])
        c = plsc.load_gather(obs_v, [rsel + 1])
        idx_v[g, :] = r * _N_COLS + c

    bufs = (buf0, buf1)
    sems = (sem0, sem1)

    def dma(g, p):
        return pltpu.make_async_copy(tab_hbm.at[idx_v.at[g]], bufs[p],
                                     sems[p])

    neg_inf = jnp.full((_L,), -jnp.inf, jnp.float32)
    zeros = jnp.zeros((_L,), jnp.int32)

    def compute_group(g, p):
        buf = bufs[p]
        dma(g, p).wait()

        for rb in range(_K // _NROWCHAIN):
            rows = [rb * _NROWCHAIN + k for k in range(_NROWCHAIN)]

            def step_iter(i, carry, rows=rows, buf=buf):
                accs, cv = carry
                accs = list(accs)
                cs = i * (_UNROLL * _L)
                for u in range(_UNROLL):
                    st = cs + u * _L
                    for k in range(_NROWCHAIN):
                        bv, bi = accs[k]
                        v = buf[rows[k], pl.ds(st, _L)]
                        m = v > bv
                        accs[k] = (jnp.where(m, v, bv),
                                   jnp.where(m, cv, bi))
                    cv = cv + _L
                return tuple(accs), cv

            init = (tuple((neg_inf, zeros) for _ in range(_NROWCHAIN)),
                    iota)
            fin, _ = lax.fori_loop(0, _STEPS // _UNROLL, step_iter, init)
            for k in range(_NROWCHAIN):
                bv, bi = fin[k]
                cval_v[rows[k], 0:_L] = bv
                cidx_v[rows[k], 0:_L] = bi

        bv = bi = None
        for c in range(_L):
            cc = jnp.full((_L,), c, jnp.int32)
            v = plsc.load_gather(cval_v, [iota, cc])
            ii = plsc.load_gather(cidx_v, [iota, cc])
            if c == 0:
                bv, bi = v, ii
            else:
                m = (v > bv) | ((v == bv) & (ii < bi))
                bv = jnp.where(m, v, bv)
                bi = jnp.where(m, ii, bi)
        plsc.store_scatter(res_v, [g * _L + iota], bi)

    dma(0, 0).start()
    dma(1, 1).start()

    def outer(t, carry):
        g2 = t * 2
        compute_group(g2, 0)

        @pl.when(g2 + 2 < _NG)
        def _():
            dma(g2 + 2, 0).start()

        compute_group(g2 + 1, 1)

        @pl.when(g2 + 3 < _NG)
        def _():
            dma(g2 + 3, 1).start()

        return carry

    lax.fori_loop(0, _NG // 2, outer, 0)

    pltpu.sync_copy(res_v, out_hbm.at[pl.ds(wid * _BPW, _BPW)])


def _run(obs, tab):
    fn = pl.kernel(
        _sc_body,
        out_type=jax.ShapeDtypeStruct((_BATCH,), jnp.int32),
        mesh=plsc.VectorSubcoreMesh(core_axis_name="c", subcore_axis_name="s"),
        compiler_params=pltpu.CompilerParams(needs_layout_passes=False),
        scratch_types=[
            pltpu.VMEM((_BPW * 2,), jnp.int32),
            pltpu.VMEM((_NG, _L), jnp.int32),
            pltpu.VMEM((_K, _N_ACT), jnp.float32),
            pltpu.VMEM((_K, _N_ACT), jnp.float32),
            pltpu.VMEM((_K, 17), jnp.float32),
            pltpu.VMEM((_K, 17), jnp.int32),
            pltpu.VMEM((_BPW,), jnp.int32),
            pltpu.SemaphoreType.DMA,
            pltpu.SemaphoreType.DMA,
        ],
    )
    return fn(obs, tab)


def kernel(observation, q_table):
    obs = observation.astype(jnp.int32).reshape(_BATCH * 2)
    tab = q_table.reshape(_N_ROWS * _N_COLS, _N_ACT)
    return _run(obs, tab)
